# baseline (speedup 1.0000x reference)
import functools

import jax
import jax.numpy as jnp
from jax import lax
from jax.experimental import pallas as pl
from jax.experimental.pallas import tpu as pltpu
from jax.experimental.pallas import tpu_sc as plsc

CYCLE = 3
D = 128
BATCH = 16384
HIST = 200
NTOT = BATCH * HIST

NC = 2
NS = 16
NW = NC * NS
PER_W = NTOT // NW

CH = 128
BSUB = 16
BLK = BSUB * CH
NBLK = PER_W // BLK
SROW_W = PER_W // CH
RING = 4


def _body(idx_hbm, table_hbm, out_hbm, tab_sh, idxb, rows,
          is0, is1, gs0, gs1, gs2, gs3, os0, os1):
    isem = [is0, is1]
    gsem = [gs0, gs1, gs2, gs3]
    osem = [os0, os1]

    cid = lax.axis_index("c")
    sid = lax.axis_index("s")
    wid = sid * NC + cid

    @pl.when(sid == 0)
    def _():
        pltpu.sync_copy(table_hbm, tab_sh)

    plsc.subcore_barrier()

    idx_row0 = wid * PER_W
    out_srow0 = wid * SROW_W

    def idx_src(g):
        return idx_hbm.at[pl.ds(idx_row0 + g * BLK, BLK)]

    def out_dst2(srow):
        return out_hbm.at[pl.ds(out_srow0 + srow, 2)]

    def wait_idx(bb):
        pltpu.make_async_copy(idx_src(0), idxb.at[bb], isem[bb]).wait()

    def wait_gat(p):
        pltpu.make_async_copy(
            tab_sh.at[idxb.at[0, pl.ds(0, CH)]], rows.at[p // 2, p % 2],
            gsem[p],
        ).wait()

    def wait_out(q):
        pltpu.make_async_copy(rows.at[0], out_dst2(0), osem[q]).wait()

    def pair_store(g, j):
        jm = j - 2
        pmq = ((jm % BSUB) // 2) % 2
        wait_gat(jm % RING)
        wait_gat((jm + 1) % RING)
        pltpu.async_copy(
            rows.at[pmq], out_dst2(g * BSUB + jm), osem[pmq]
        )

    pltpu.async_copy(idx_src(0), idxb.at[0], isem[0])

    def blk2(g2, carry):
        for bb in range(2):
            g = g2 * 2 + bb
            wait_idx(bb)

            def wrap(i, c):
                v = idxb[bb, pl.ds(i * 16, 16)]
                idxb[bb, pl.ds(i * 16, 16)] = lax.rem(v, CYCLE)
                return c

            lax.fori_loop(0, BLK // 16, wrap, 0)

            for j in range(BSUB):
                pp, s = (j // 2) % 2, j % 2
                if j % 2 == 0:
                    if j < 4:
                        @pl.when(g > 0)
                        def _(q=pp):
                            wait_out(q)
                    else:
                        wait_out(pp)
                pltpu.async_copy(
                    tab_sh.at[idxb.at[bb, pl.ds(j * CH, CH)]],
                    rows.at[pp, s], gsem[j % RING],
                )
                if j == 0:
                    @pl.when(g > 0)
                    def _():
                        pair_store(g, 0)
                elif j % 2 == 0:
                    pair_store(g, j)
                if j == 1:
                    @pl.when(g < NBLK - 1)
                    def _():
                        pltpu.async_copy(
                            idx_src(g + 1), idxb.at[1 - bb], isem[1 - bb]
                        )
        return carry

    lax.fori_loop(0, NBLK // 2, blk2, 0)

    wait_gat((BSUB - 2) % RING)
    wait_gat((BSUB - 1) % RING)
    pltpu.async_copy(
        rows.at[1], out_dst2((NBLK - 1) * BSUB + BSUB - 2), osem[1]
    )
    for q in range(2):
        wait_out(q)


@jax.jit
def _run(idx2, table):
    mesh = plsc.VectorSubcoreMesh(core_axis_name="c", subcore_axis_name="s")
    return pl.kernel(
        _body,
        out_type=jax.ShapeDtypeStruct((NTOT // CH, CH, D), jnp.float32),
        mesh=mesh,
        scratch_types=[
            pltpu.VMEM_SHARED((CYCLE, D), jnp.float32),
            pltpu.VMEM((2, BLK), jnp.int32),
            pltpu.VMEM((2, 2, CH, D), jnp.float32),
            pltpu.SemaphoreType.DMA,
            pltpu.SemaphoreType.DMA,
            pltpu.SemaphoreType.DMA,
            pltpu.SemaphoreType.DMA,
            pltpu.SemaphoreType.DMA,
            pltpu.SemaphoreType.DMA,
            pltpu.SemaphoreType.DMA,
            pltpu.SemaphoreType.DMA,
        ],
    )(idx2, table)


def kernel(idx, table):
    out = _run(idx.reshape(NTOT), table)
    return out.reshape(BATCH, HIST, D)


# speedup vs baseline: 1.0789x; 1.0789x over previous
import functools

import jax
import jax.numpy as jnp
from jax import lax
from jax.experimental import pallas as pl
from jax.experimental.pallas import tpu as pltpu
from jax.experimental.pallas import tpu_sc as plsc

CYCLE = 3
D = 128
BATCH = 16384
HIST = 200
NTOT = BATCH * HIST

NC = 2
NS = 16
NW = NC * NS
PER_W = NTOT // NW

CH = 128
BSUB = 8
BLK = BSUB * CH
NBLK = PER_W // BLK
SROW_W = PER_W // CH
RING = 4


def _body(idx_hbm, table_hbm, out_hbm, tab_sh, idxb, rows,
          is0, is1, gs0, gs1, gs2, gs3, os0, os1):
    isem = [is0, is1]
    gsem = [gs0, gs1, gs2, gs3]
    osem = [os0, os1]

    cid = lax.axis_index("c")
    sid = lax.axis_index("s")
    wid = sid * NC + cid

    @pl.when(sid == 0)
    def _():
        pltpu.sync_copy(table_hbm, tab_sh)

    plsc.subcore_barrier()

    idx_row0 = wid * PER_W
    out_srow0 = wid * SROW_W

    def idx_src(g):
        return idx_hbm.at[pl.ds(idx_row0 + g * BLK, BLK)]

    def out_dst2(srow):
        return out_hbm.at[pl.ds(out_srow0 + srow, 2)]

    def wait_idx(bb):
        pltpu.make_async_copy(idx_src(0), idxb.at[bb], isem[bb]).wait()

    def wait_gat(p):
        pltpu.make_async_copy(
            tab_sh.at[idxb.at[0, pl.ds(0, CH)]], rows.at[p // 2, p % 2],
            gsem[p],
        ).wait()

    def wait_out(q):
        pltpu.make_async_copy(rows.at[0], out_dst2(0), osem[q]).wait()

    def pair_store(g, j):
        jm = j - 2
        pmq = ((jm % BSUB) // 2) % 2
        wait_gat(jm % RING)
        wait_gat((jm + 1) % RING)
        pltpu.async_copy(
            rows.at[pmq], out_dst2(g * BSUB + jm), osem[pmq]
        )

    pltpu.async_copy(idx_src(0), idxb.at[0], isem[0])

    def blk2(g2, carry):
        for bb in range(2):
            g = g2 * 2 + bb
            wait_idx(bb)

            def wrap_slice(sl):
                def wrap(i, c):
                    o = sl * CH + i * 16
                    idxb[bb, pl.ds(o, 16)] = lax.rem(
                        idxb[bb, pl.ds(o, 16)], CYCLE
                    )
                    return c

                lax.fori_loop(0, CH // 16, wrap, 0)

            wrap_slice(0)
            wrap_slice(1)

            for j in range(BSUB):
                pp, s = (j // 2) % 2, j % 2
                if j % 2 == 0:
                    if j < 4:
                        @pl.when(g > 0)
                        def _(q=pp):
                            wait_out(q)
                    else:
                        wait_out(pp)
                pltpu.async_copy(
                    tab_sh.at[idxb.at[bb, pl.ds(j * CH, CH)]],
                    rows.at[pp, s], gsem[j % RING],
                )
                if j + 2 < BSUB:
                    wrap_slice(j + 2)
                if j == 0:
                    @pl.when(g > 0)
                    def _():
                        pair_store(g, 0)
                elif j % 2 == 0:
                    pair_store(g, j)
                if j == 1:
                    @pl.when(g < NBLK - 1)
                    def _():
                        pltpu.async_copy(
                            idx_src(g + 1), idxb.at[1 - bb], isem[1 - bb]
                        )
        return carry

    lax.fori_loop(0, NBLK // 2, blk2, 0)

    wait_gat((BSUB - 2) % RING)
    wait_gat((BSUB - 1) % RING)
    pltpu.async_copy(
        rows.at[1], out_dst2((NBLK - 1) * BSUB + BSUB - 2), osem[1]
    )
    for q in range(2):
        wait_out(q)


@jax.jit
def _run(idx2, table):
    mesh = plsc.VectorSubcoreMesh(core_axis_name="c", subcore_axis_name="s")
    return pl.kernel(
        _body,
        out_type=jax.ShapeDtypeStruct((NTOT // CH, CH, D), jnp.float32),
        mesh=mesh,
        scratch_types=[
            pltpu.VMEM_SHARED((CYCLE, D), jnp.float32),
            pltpu.VMEM((2, BLK), jnp.int32),
            pltpu.VMEM((2, 2, CH, D), jnp.float32),
            pltpu.SemaphoreType.DMA,
            pltpu.SemaphoreType.DMA,
            pltpu.SemaphoreType.DMA,
            pltpu.SemaphoreType.DMA,
            pltpu.SemaphoreType.DMA,
            pltpu.SemaphoreType.DMA,
            pltpu.SemaphoreType.DMA,
            pltpu.SemaphoreType.DMA,
        ],
    )(idx2, table)


def kernel(idx, table):
    out = _run(idx.reshape(NTOT), table)
    return out.reshape(BATCH, HIST, D)
